# baseline (speedup 1.0000x reference)
import functools

import jax
import jax.numpy as jnp
from jax.experimental import pallas as pl
from jax.experimental.pallas import tpu as pltpu

D = 128
_BN_INV = 1.0 / (1.0 + 1e-5) ** 0.5


def _gin_mlp_body(x_ref, agg_ref, w1_ref, bb_ref, s_ref, w2_ref, b2_ref, o_ref):
    h = x_ref[...] + agg_ref[...]
    t = jnp.dot(h, w1_ref[...], preferred_element_type=jnp.float32)
    t = jax.nn.relu(t * s_ref[...] + bb_ref[...])
    t = jnp.dot(t, w2_ref[...], preferred_element_type=jnp.float32)
    o_ref[...] = jax.nn.relu(t + b2_ref[...])


def _gin_mlp(x, agg, p, block_rows=1024):
    r = x.shape[0]
    s = (p["g"] * _BN_INV).reshape(1, D)
    bb = (p["b1"] * p["g"] * _BN_INV + p["be"]).reshape(1, D)
    b2 = p["b2"].reshape(1, D)
    grid = (pl.cdiv(r, block_rows),)
    return pl.pallas_call(
        _gin_mlp_body,
        grid=grid,
        in_specs=[
            pl.BlockSpec((block_rows, D), lambda i: (i, 0)),
            pl.BlockSpec((block_rows, D), lambda i: (i, 0)),
            pl.BlockSpec((D, D), lambda i: (0, 0)),
            pl.BlockSpec((1, D), lambda i: (0, 0)),
            pl.BlockSpec((1, D), lambda i: (0, 0)),
            pl.BlockSpec((D, D), lambda i: (0, 0)),
            pl.BlockSpec((1, D), lambda i: (0, 0)),
        ],
        out_specs=pl.BlockSpec((block_rows, D), lambda i: (i, 0)),
        out_shape=jax.ShapeDtypeStruct((r, D), jnp.float32),
    )(x, agg, p["w1"], bb, s, p["w2"], b2)


def _pair_mlp_body(a_ref, b_ref, w1_ref, bb_ref, s_ref, w2_ref, b2_ref, o_ref):
    ta = jnp.dot(a_ref[...], w1_ref[...], preferred_element_type=jnp.float32)
    tb = jnp.dot(b_ref[...], w1_ref[...], preferred_element_type=jnp.float32)
    ta = jax.nn.relu(ta * s_ref[...] + bb_ref[...])
    tb = jax.nn.relu(tb * s_ref[...] + bb_ref[...])
    t = jnp.dot(ta + tb, w2_ref[...], preferred_element_type=jnp.float32)
    o_ref[...] = t + b2_ref[...]


def _pair_mlp(a, b, p, block_rows=1024):
    r = a.shape[0]
    s = (p["g"] * _BN_INV).reshape(1, D)
    bb = (p["b1"] * p["g"] * _BN_INV + p["be"]).reshape(1, D)
    b2 = (2.0 * p["b2"]).reshape(1, D)
    grid = (pl.cdiv(r, block_rows),)
    return pl.pallas_call(
        _pair_mlp_body,
        grid=grid,
        in_specs=[
            pl.BlockSpec((block_rows, D), lambda i: (i, 0)),
            pl.BlockSpec((block_rows, D), lambda i: (i, 0)),
            pl.BlockSpec((D, D), lambda i: (0, 0)),
            pl.BlockSpec((1, D), lambda i: (0, 0)),
            pl.BlockSpec((1, D), lambda i: (0, 0)),
            pl.BlockSpec((D, D), lambda i: (0, 0)),
            pl.BlockSpec((1, D), lambda i: (0, 0)),
        ],
        out_specs=pl.BlockSpec((block_rows, D), lambda i: (i, 0)),
        out_shape=jax.ShapeDtypeStruct((r, D), jnp.float32),
    )(a, b, p["w1"], bb, s, p["w2"], b2)


def _linear_body(x_ref, w_ref, b_ref, o_ref):
    t = jnp.dot(x_ref[...], w_ref[...], preferred_element_type=jnp.float32)
    o_ref[...] = t + b_ref[...]


def _linear(x, w, b, block_rows=1024):
    r = x.shape[0]
    grid = (pl.cdiv(r, block_rows),)
    return pl.pallas_call(
        _linear_body,
        grid=grid,
        in_specs=[
            pl.BlockSpec((block_rows, D), lambda i: (i, 0)),
            pl.BlockSpec((D, D), lambda i: (0, 0)),
            pl.BlockSpec((1, D), lambda i: (0, 0)),
        ],
        out_specs=pl.BlockSpec((block_rows, D), lambda i: (i, 0)),
        out_shape=jax.ShapeDtypeStruct((r, D), jnp.float32),
    )(x, w, b.reshape(1, D))


def kernel(x_original, edge_index, edge_index_original, edge_pairs, params):
    ei2 = edge_index.astype(jnp.int32)
    eio = edge_index_original.astype(jnp.int32)
    ep = edge_pairs.astype(jnp.int32)

    x0 = x_original
    for p in params["init"]:
        agg = jnp.zeros_like(x0).at[eio[1]].add(x0[eio[0]])
        x0 = _gin_mlp(x0, agg, p)

    a = x0[ep[:, 0]]
    b = x0[ep[:, 1]]
    x = _pair_mlp(a, b, params["mlp"])

    for p in params["gin"]:
        agg = jnp.zeros_like(x).at[ei2[1]].add(x[ei2[0]])
        x = _gin_mlp(x, agg, p)

    return _linear(x, params["wo"], params["bo"])


# speedup vs baseline: 1.0208x; 1.0208x over previous
import functools

import jax
import jax.numpy as jnp
from jax import lax
from jax.experimental import pallas as pl
from jax.experimental.pallas import tpu as pltpu
from jax.experimental.pallas import tpu_sc as plsc

D = 128
_NC, _NS, _L = 2, 16, 16
_NW = _NC * _NS
_BN_INV = 1.0 / (1.0 + 1e-5) ** 0.5


def _gin_mlp_body(x_ref, agg_ref, w1_ref, bb_ref, s_ref, w2_ref, b2_ref, o_ref):
    h = x_ref[...] + agg_ref[...]
    t = jnp.dot(h, w1_ref[...], preferred_element_type=jnp.float32)
    t = jax.nn.relu(t * s_ref[...] + bb_ref[...])
    t = jnp.dot(t, w2_ref[...], preferred_element_type=jnp.float32)
    o_ref[...] = jax.nn.relu(t + b2_ref[...])


def _gin_mlp(x, agg, p, block_rows=1024):
    r = x.shape[0]
    s = (p["g"] * _BN_INV).reshape(1, D)
    bb = (p["b1"] * p["g"] * _BN_INV + p["be"]).reshape(1, D)
    b2 = p["b2"].reshape(1, D)
    grid = (pl.cdiv(r, block_rows),)
    return pl.pallas_call(
        _gin_mlp_body,
        grid=grid,
        in_specs=[
            pl.BlockSpec((block_rows, D), lambda i: (i, 0)),
            pl.BlockSpec((block_rows, D), lambda i: (i, 0)),
            pl.BlockSpec((D, D), lambda i: (0, 0)),
            pl.BlockSpec((1, D), lambda i: (0, 0)),
            pl.BlockSpec((1, D), lambda i: (0, 0)),
            pl.BlockSpec((D, D), lambda i: (0, 0)),
            pl.BlockSpec((1, D), lambda i: (0, 0)),
        ],
        out_specs=pl.BlockSpec((block_rows, D), lambda i: (i, 0)),
        out_shape=jax.ShapeDtypeStruct((r, D), jnp.float32),
    )(x, agg, p["w1"], bb, s, p["w2"], b2)


def _pair_gather(x0, ep_t):
    e = ep_t.shape[1]
    ep_flat = ep_t.reshape(2 * e)
    rows_per = (2 * e) // _NW
    kwin = 80
    nwin = rows_per // kwin
    mesh = plsc.VectorSubcoreMesh(core_axis_name="c", subcore_axis_name="s")

    @functools.partial(
        pl.kernel,
        out_type=jax.ShapeDtypeStruct((2, e, D), jnp.float32),
        mesh=mesh,
        scratch_types=[
            pltpu.VMEM((rows_per,), jnp.int32),
            pltpu.VMEM((kwin, D), jnp.float32),
            pltpu.SemaphoreType.DMA,
        ],
    )
    def gk(x_hbm, ep_hbm, out_hbm, idx_v, buf, sem):
        w = lax.axis_index("c") * _NS + lax.axis_index("s")
        part = w // _NS
        rowbase = (w % _NS) * rows_per
        pltpu.sync_copy(ep_hbm.at[pl.ds(w * rows_per, rows_per)], idx_v)

        @pl.loop(0, nwin)
        def _win(j):
            idx = idx_v.at[pl.ds(j * kwin, kwin)]
            pltpu.async_copy(x_hbm.at[idx], buf, sem).wait()
            pltpu.sync_copy(buf, out_hbm.at[part, pl.ds(rowbase + j * kwin, kwin)])

    return gk(x0, ep_flat)


def _pair_mlp_body(g_ref, w1_ref, bb_ref, s_ref, w2_ref, b2_ref, o_ref):
    ta = jnp.dot(g_ref[0], w1_ref[...], preferred_element_type=jnp.float32)
    tb = jnp.dot(g_ref[1], w1_ref[...], preferred_element_type=jnp.float32)
    ta = jax.nn.relu(ta * s_ref[...] + bb_ref[...])
    tb = jax.nn.relu(tb * s_ref[...] + bb_ref[...])
    t = jnp.dot(ta + tb, w2_ref[...], preferred_element_type=jnp.float32)
    o_ref[...] = t + b2_ref[...]


def _pair_mlp(g, p, block_rows=1024):
    r = g.shape[1]
    s = (p["g"] * _BN_INV).reshape(1, D)
    bb = (p["b1"] * p["g"] * _BN_INV + p["be"]).reshape(1, D)
    b2 = (2.0 * p["b2"]).reshape(1, D)
    grid = (pl.cdiv(r, block_rows),)
    return pl.pallas_call(
        _pair_mlp_body,
        grid=grid,
        in_specs=[
            pl.BlockSpec((2, block_rows, D), lambda i: (0, i, 0)),
            pl.BlockSpec((D, D), lambda i: (0, 0)),
            pl.BlockSpec((1, D), lambda i: (0, 0)),
            pl.BlockSpec((1, D), lambda i: (0, 0)),
            pl.BlockSpec((D, D), lambda i: (0, 0)),
            pl.BlockSpec((1, D), lambda i: (0, 0)),
        ],
        out_specs=pl.BlockSpec((block_rows, D), lambda i: (i, 0)),
        out_shape=jax.ShapeDtypeStruct((r, D), jnp.float32),
    )(g, p["w1"], bb, s, p["w2"], b2)


def _linear_body(x_ref, w_ref, b_ref, o_ref):
    t = jnp.dot(x_ref[...], w_ref[...], preferred_element_type=jnp.float32)
    o_ref[...] = t + b_ref[...]


def _linear(x, w, b, block_rows=1024):
    r = x.shape[0]
    grid = (pl.cdiv(r, block_rows),)
    return pl.pallas_call(
        _linear_body,
        grid=grid,
        in_specs=[
            pl.BlockSpec((block_rows, D), lambda i: (i, 0)),
            pl.BlockSpec((D, D), lambda i: (0, 0)),
            pl.BlockSpec((1, D), lambda i: (0, 0)),
        ],
        out_specs=pl.BlockSpec((block_rows, D), lambda i: (i, 0)),
        out_shape=jax.ShapeDtypeStruct((r, D), jnp.float32),
    )(x, w, b.reshape(1, D))


def kernel(x_original, edge_index, edge_index_original, edge_pairs, params):
    ei2 = edge_index.astype(jnp.int32)
    eio = edge_index_original.astype(jnp.int32)
    ep = edge_pairs.astype(jnp.int32)

    x0 = x_original
    for p in params["init"]:
        agg = jnp.zeros_like(x0).at[eio[1]].add(x0[eio[0]])
        x0 = _gin_mlp(x0, agg, p)

    g = _pair_gather(x0, ep.T)
    x = _pair_mlp(g, params["mlp"])

    for p in params["gin"]:
        agg = jnp.zeros_like(x).at[ei2[1]].add(x[ei2[0]])
        x = _gin_mlp(x, agg, p)

    return _linear(x, params["wo"], params["bo"])


# speedup vs baseline: 1.1370x; 1.1139x over previous
import functools

import jax
import jax.numpy as jnp
from jax import lax
from jax.experimental import pallas as pl
from jax.experimental.pallas import tpu as pltpu
from jax.experimental.pallas import tpu_sc as plsc

D = 128
_NC, _NS, _L = 2, 16, 16
_NW = _NC * _NS
_BN_INV = 1.0 / (1.0 + 1e-5) ** 0.5


def _scatter_add_small(x, src, dst, n):
    e = src.shape[0]
    kwin = 128
    nwin = e // kwin
    rows_a = ((n // _NS) + 7) // 8 * 8
    rows_last = n - rows_a * (_NS - 1)
    assert rows_last > 0 and rows_last % 8 == 0
    mesh = plsc.VectorSubcoreMesh(core_axis_name="c", subcore_axis_name="s")
    zeros = jnp.zeros((rows_a, D), jnp.float32)

    @functools.partial(
        pl.kernel,
        out_type=jax.ShapeDtypeStruct((2, n, D), jnp.float32),
        mesh=mesh,
        scratch_types=[
            pltpu.VMEM((kwin,), jnp.int32),
            pltpu.VMEM((kwin,), jnp.int32),
            pltpu.VMEM((kwin, D), jnp.float32),
            pltpu.VMEM_SHARED((n, D), jnp.float32),
            pltpu.SemaphoreType.DMA,
        ],
    )
    def sk(x_hbm, src_hbm, dst_hbm, z_hbm, out_hbm, srcw, dstw, buf, acc, sem):
        c = lax.axis_index("c")
        s = lax.axis_index("s")

        @pl.when(s < _NS - 1)
        def _z0():
            pltpu.sync_copy(z_hbm, acc.at[pl.ds(s * rows_a, rows_a)])

        @pl.when(s == _NS - 1)
        def _z1():
            pltpu.sync_copy(z_hbm.at[pl.ds(0, rows_last)],
                            acc.at[pl.ds((_NS - 1) * rows_a, rows_last)])

        plsc.subcore_barrier()

        @pl.loop(c * _NS + s, nwin, step=_NW)
        def _win(j):
            pltpu.sync_copy(src_hbm.at[pl.ds(j * kwin, kwin)], srcw)
            pltpu.sync_copy(dst_hbm.at[pl.ds(j * kwin, kwin)], dstw)
            pltpu.async_copy(x_hbm.at[srcw], buf, sem).wait()
            pltpu.sync_copy(buf, acc.at[dstw], add=True)

        plsc.subcore_barrier()

        @pl.when(s < _NS - 1)
        def _w0():
            pltpu.sync_copy(acc.at[pl.ds(s * rows_a, rows_a)],
                            out_hbm.at[c, pl.ds(s * rows_a, rows_a)])

        @pl.when(s == _NS - 1)
        def _w1():
            pltpu.sync_copy(acc.at[pl.ds((_NS - 1) * rows_a, rows_last)],
                            out_hbm.at[c, pl.ds((_NS - 1) * rows_a, rows_last)])

    return sk(x, src, dst, zeros)


def _gin_mlp_body(x_ref, agg_ref, w1_ref, bb_ref, s_ref, w2_ref, b2_ref, o_ref):
    h = x_ref[...] + agg_ref[0]
    for p in range(1, agg_ref.shape[0]):
        h = h + agg_ref[p]
    t = jnp.dot(h, w1_ref[...], preferred_element_type=jnp.float32)
    t = jax.nn.relu(t * s_ref[...] + bb_ref[...])
    t = jnp.dot(t, w2_ref[...], preferred_element_type=jnp.float32)
    o_ref[...] = jax.nn.relu(t + b2_ref[...])


def _gin_mlp(x, agg, p, block_rows=1024):
    r = x.shape[0]
    nparts = agg.shape[0]
    s = (p["g"] * _BN_INV).reshape(1, D)
    bb = (p["b1"] * p["g"] * _BN_INV + p["be"]).reshape(1, D)
    b2 = p["b2"].reshape(1, D)
    grid = (pl.cdiv(r, block_rows),)
    return pl.pallas_call(
        _gin_mlp_body,
        grid=grid,
        in_specs=[
            pl.BlockSpec((block_rows, D), lambda i: (i, 0)),
            pl.BlockSpec((nparts, block_rows, D), lambda i: (0, i, 0)),
            pl.BlockSpec((D, D), lambda i: (0, 0)),
            pl.BlockSpec((1, D), lambda i: (0, 0)),
            pl.BlockSpec((1, D), lambda i: (0, 0)),
            pl.BlockSpec((D, D), lambda i: (0, 0)),
            pl.BlockSpec((1, D), lambda i: (0, 0)),
        ],
        out_specs=pl.BlockSpec((block_rows, D), lambda i: (i, 0)),
        out_shape=jax.ShapeDtypeStruct((r, D), jnp.float32),
    )(x, agg, p["w1"], bb, s, p["w2"], b2)


def _pair_gather(x0, ep_t):
    e = ep_t.shape[1]
    ep_flat = ep_t.reshape(2 * e)
    rows_per = (2 * e) // _NW
    kwin = 80
    nwin = rows_per // kwin
    mesh = plsc.VectorSubcoreMesh(core_axis_name="c", subcore_axis_name="s")

    @functools.partial(
        pl.kernel,
        out_type=jax.ShapeDtypeStruct((2, e, D), jnp.float32),
        mesh=mesh,
        scratch_types=[
            pltpu.VMEM((rows_per,), jnp.int32),
            pltpu.VMEM((kwin, D), jnp.float32),
            pltpu.SemaphoreType.DMA,
        ],
    )
    def gk(x_hbm, ep_hbm, out_hbm, idx_v, buf, sem):
        w = lax.axis_index("c") * _NS + lax.axis_index("s")
        part = w // _NS
        rowbase = (w % _NS) * rows_per
        pltpu.sync_copy(ep_hbm.at[pl.ds(w * rows_per, rows_per)], idx_v)

        @pl.loop(0, nwin)
        def _win(j):
            idx = idx_v.at[pl.ds(j * kwin, kwin)]
            pltpu.async_copy(x_hbm.at[idx], buf, sem).wait()
            pltpu.sync_copy(buf, out_hbm.at[part, pl.ds(rowbase + j * kwin, kwin)])

    return gk(x0, ep_flat)


def _pair_mlp_body(g_ref, w1_ref, bb_ref, s_ref, w2_ref, b2_ref, o_ref):
    ta = jnp.dot(g_ref[0], w1_ref[...], preferred_element_type=jnp.float32)
    tb = jnp.dot(g_ref[1], w1_ref[...], preferred_element_type=jnp.float32)
    ta = jax.nn.relu(ta * s_ref[...] + bb_ref[...])
    tb = jax.nn.relu(tb * s_ref[...] + bb_ref[...])
    t = jnp.dot(ta + tb, w2_ref[...], preferred_element_type=jnp.float32)
    o_ref[...] = t + b2_ref[...]


def _pair_mlp(g, p, block_rows=1024):
    r = g.shape[1]
    s = (p["g"] * _BN_INV).reshape(1, D)
    bb = (p["b1"] * p["g"] * _BN_INV + p["be"]).reshape(1, D)
    b2 = (2.0 * p["b2"]).reshape(1, D)
    grid = (pl.cdiv(r, block_rows),)
    return pl.pallas_call(
        _pair_mlp_body,
        grid=grid,
        in_specs=[
            pl.BlockSpec((2, block_rows, D), lambda i: (0, i, 0)),
            pl.BlockSpec((D, D), lambda i: (0, 0)),
            pl.BlockSpec((1, D), lambda i: (0, 0)),
            pl.BlockSpec((1, D), lambda i: (0, 0)),
            pl.BlockSpec((D, D), lambda i: (0, 0)),
            pl.BlockSpec((1, D), lambda i: (0, 0)),
        ],
        out_specs=pl.BlockSpec((block_rows, D), lambda i: (i, 0)),
        out_shape=jax.ShapeDtypeStruct((r, D), jnp.float32),
    )(g, p["w1"], bb, s, p["w2"], b2)


def _linear_body(x_ref, w_ref, b_ref, o_ref):
    t = jnp.dot(x_ref[...], w_ref[...], preferred_element_type=jnp.float32)
    o_ref[...] = t + b_ref[...]


def _linear(x, w, b, block_rows=1024):
    r = x.shape[0]
    grid = (pl.cdiv(r, block_rows),)
    return pl.pallas_call(
        _linear_body,
        grid=grid,
        in_specs=[
            pl.BlockSpec((block_rows, D), lambda i: (i, 0)),
            pl.BlockSpec((D, D), lambda i: (0, 0)),
            pl.BlockSpec((1, D), lambda i: (0, 0)),
        ],
        out_specs=pl.BlockSpec((block_rows, D), lambda i: (i, 0)),
        out_shape=jax.ShapeDtypeStruct((r, D), jnp.float32),
    )(x, w, b.reshape(1, D))


def kernel(x_original, edge_index, edge_index_original, edge_pairs, params):
    ei2 = edge_index.astype(jnp.int32)
    eio = edge_index_original.astype(jnp.int32)
    ep = edge_pairs.astype(jnp.int32)

    x0 = x_original
    for p in params["init"]:
        agg = _scatter_add_small(x0, eio[0], eio[1], x0.shape[0])
        x0 = _gin_mlp(x0, agg, p)

    g = _pair_gather(x0, ep.T)
    x = _pair_mlp(g, params["mlp"])

    for p in params["gin"]:
        agg = jnp.zeros_like(x).at[ei2[1]].add(x[ei2[0]])
        x = _gin_mlp(x, agg[None], p)

    return _linear(x, params["wo"], params["bo"])


# speedup vs baseline: 2.6316x; 2.3145x over previous
import functools

import jax
import jax.numpy as jnp
from jax import lax
from jax.experimental import pallas as pl
from jax.experimental.pallas import tpu as pltpu
from jax.experimental.pallas import tpu_sc as plsc

D = 128
_NC, _NS, _L = 2, 16, 16
_NW = _NC * _NS
_BN_INV = 1.0 / (1.0 + 1e-5) ** 0.5


def _scatter_add_small(x, src, dst, n):
    e = src.shape[0]
    kwin = 128
    nwin = e // kwin
    rows_a = ((n // _NS) + 7) // 8 * 8
    rows_last = n - rows_a * (_NS - 1)
    assert rows_last > 0 and rows_last % 8 == 0
    mesh = plsc.VectorSubcoreMesh(core_axis_name="c", subcore_axis_name="s")
    zeros = jnp.zeros((rows_a, D), jnp.float32)

    @functools.partial(
        pl.kernel,
        out_type=jax.ShapeDtypeStruct((2, n, D), jnp.float32),
        mesh=mesh,
        scratch_types=[
            pltpu.VMEM((kwin,), jnp.int32),
            pltpu.VMEM((kwin,), jnp.int32),
            pltpu.VMEM((kwin, D), jnp.float32),
            pltpu.VMEM_SHARED((n, D), jnp.float32),
            pltpu.SemaphoreType.DMA,
        ],
    )
    def sk(x_hbm, src_hbm, dst_hbm, z_hbm, out_hbm, srcw, dstw, buf, acc, sem):
        c = lax.axis_index("c")
        s = lax.axis_index("s")

        @pl.when(s < _NS - 1)
        def _z0():
            pltpu.sync_copy(z_hbm, acc.at[pl.ds(s * rows_a, rows_a)])

        @pl.when(s == _NS - 1)
        def _z1():
            pltpu.sync_copy(z_hbm.at[pl.ds(0, rows_last)],
                            acc.at[pl.ds((_NS - 1) * rows_a, rows_last)])

        plsc.subcore_barrier()

        @pl.loop(c * _NS + s, nwin, step=_NW)
        def _win(j):
            pltpu.sync_copy(src_hbm.at[pl.ds(j * kwin, kwin)], srcw)
            pltpu.sync_copy(dst_hbm.at[pl.ds(j * kwin, kwin)], dstw)
            pltpu.async_copy(x_hbm.at[srcw], buf, sem).wait()
            pltpu.sync_copy(buf, acc.at[dstw], add=True)

        plsc.subcore_barrier()

        @pl.when(s < _NS - 1)
        def _w0():
            pltpu.sync_copy(acc.at[pl.ds(s * rows_a, rows_a)],
                            out_hbm.at[c, pl.ds(s * rows_a, rows_a)])

        @pl.when(s == _NS - 1)
        def _w1():
            pltpu.sync_copy(acc.at[pl.ds((_NS - 1) * rows_a, rows_last)],
                            out_hbm.at[c, pl.ds((_NS - 1) * rows_a, rows_last)])

    return sk(x, src, dst, zeros)


_NB = 40
_BK = 8192
_IOTA = lambda: lax.iota(jnp.int32, _L)


def _lg_histogram(dst):
    e2 = dst.shape[0]
    per_w = e2 // _NW
    chunk = 8000
    nchunk = per_w // chunk
    mesh = plsc.VectorSubcoreMesh(core_axis_name="c", subcore_axis_name="s")

    @functools.partial(
        pl.kernel,
        out_type=jax.ShapeDtypeStruct((_NW * _NB * _L,), jnp.int32),
        mesh=mesh,
        compiler_params=pltpu.CompilerParams(needs_layout_passes=False),
        scratch_types=[
            pltpu.VMEM((chunk,), jnp.int32),
            pltpu.VMEM((_NB * _L,), jnp.int32),
        ],
    )
    def hk(dst_hbm, counts_hbm, dstv, hist):
        w = lax.axis_index("c") * _NS + lax.axis_index("s")
        zero16 = jnp.zeros((_L,), jnp.int32)

        @pl.loop(0, _NB)
        def _z(i):
            hist[pl.ds(i * _L, _L)] = zero16

        ones = jnp.ones((_L,), jnp.int32)
        iota = _IOTA()

        @pl.loop(0, nchunk)
        def _c(j):
            pltpu.sync_copy(dst_hbm.at[pl.ds(w * per_w + j * chunk, chunk)], dstv)

            @pl.loop(0, chunk // _L)
            def _v(k):
                d = dstv[pl.ds(k * _L, _L)]
                b = lax.shift_right_logical(d, 13)
                addr = b * _L + iota
                plsc.addupdate_scatter(hist, [addr], ones)

        pltpu.sync_copy(hist, counts_hbm.at[pl.ds(w * _NB * _L, _NB * _L)])

    return hk(dst)


def _lg_permute(src, dst, counts):
    e2 = src.shape[0]
    e2p = (e2 // 128 + 32) * 128
    per_w = e2 // _NW
    win = 16000
    nwin = per_w // win
    ncnt = _NW * _NB * _L
    mesh = plsc.VectorSubcoreMesh(core_axis_name="c", subcore_axis_name="s")

    @functools.partial(
        pl.kernel,
        out_type=(
            jax.ShapeDtypeStruct((e2p,), jnp.int32),
            jax.ShapeDtypeStruct((e2p,), jnp.int32),
            jax.ShapeDtypeStruct((64,), jnp.int32),
        ),
        mesh=mesh,
        compiler_params=pltpu.CompilerParams(needs_layout_passes=False),
        scratch_types=[
            pltpu.VMEM((ncnt,), jnp.int32),
            pltpu.VMEM((ncnt,), jnp.int32),
            pltpu.VMEM((_NB * _L,), jnp.int32),
            pltpu.VMEM((win,), jnp.int32),
            pltpu.VMEM((win,), jnp.int32),
            pltpu.VMEM((win,), jnp.int32),
            pltpu.VMEM((win // 128, 128), jnp.int32),
            pltpu.VMEM((64,), jnp.int32),
            pltpu.SemaphoreType.DMA,
        ],
    )
    def pk(src_hbm, dst_hbm, counts_hbm, ss_hbm, ls_hbm, offs_hbm,
           counts_v, start_v, nxt, srcv, dstv, ldstv, posv, offs_v, sem):
        w = lax.axis_index("c") * _NS + lax.axis_index("s")
        iota = _IOTA()
        pltpu.sync_copy(counts_hbm, counts_v)

        def _scan(i, carry):
            b = i // _NW
            w2 = i % _NW
            v = counts_v[pl.ds(w2 * _NB * _L + b * _L, _L)]
            cs = plsc.cumsum(v)
            start_v[pl.ds(i * _L, _L)] = (cs - v) + carry
            return carry + jnp.sum(v)

        lax.fori_loop(0, _NB * _NW, _scan, jnp.int32(0))

        @pl.when(w == 0)
        def _offs():
            for i in range(4):
                bid = iota + i * _L
                gidx = jnp.minimum(bid * (_NW * _L), ncnt - 1)
                vals = plsc.load_gather(start_v, [gidx])
                offs_v[pl.ds(i * _L, _L)] = jnp.where(bid >= _NB, e2, vals)
            pltpu.sync_copy(offs_v, offs_hbm)

        @pl.loop(0, _NB)
        def _n(b):
            nxt[pl.ds(b * _L, _L)] = start_v[pl.ds((b * _NW + w) * _L, _L)]

        @pl.loop(0, nwin)
        def _w(j):
            base = w * per_w + j * win
            pltpu.sync_copy(src_hbm.at[pl.ds(base, win)], srcv)
            pltpu.sync_copy(dst_hbm.at[pl.ds(base, win)], dstv)

            @pl.loop(0, win // _L)
            def _v(k):
                d = dstv[pl.ds(k * _L, _L)]
                b = lax.shift_right_logical(d, 13)
                addr = b * _L + iota
                pos = plsc.load_gather(nxt, [addr])
                plsc.store_scatter(nxt, [addr], pos + 1)
                ldstv[pl.ds(k * _L, _L)] = d & 8191
                r = k // 8
                c = k % 8
                posv[r, pl.ds(c * _L, _L)] = pos

            @pl.loop(0, win // 128)
            def _s(r):
                pltpu.async_copy(srcv.at[pl.ds(r * 128, 128)],
                                 ss_hbm.at[posv.at[r]], sem).wait()
                pltpu.async_copy(ldstv.at[pl.ds(r * 128, 128)],
                                 ls_hbm.at[posv.at[r]], sem).wait()

    return pk(src, dst, counts)


def _lg_aggregate(x, ss2d, ls2d, offs):
    e = x.shape[0]
    nrows = ss2d.shape[0]
    acc_rows = _BK + 8
    zrows = acc_rows - 15 * (_BK // _NS)
    zeros = jnp.zeros((zrows, D), jnp.float32)
    mesh = plsc.VectorSubcoreMesh(core_axis_name="c", subcore_axis_name="s")

    @functools.partial(
        pl.kernel,
        out_type=jax.ShapeDtypeStruct((_NB * _BK, D), jnp.float32),
        mesh=mesh,
        compiler_params=pltpu.CompilerParams(needs_layout_passes=False),
        scratch_types=[
            pltpu.VMEM((64,), jnp.int32),
            pltpu.VMEM((8, 128), jnp.int32),
            pltpu.VMEM((8, 128), jnp.int32),
            pltpu.VMEM((128, D), jnp.float32),
            pltpu.VMEM((128, D), jnp.float32),
            pltpu.VMEM_SHARED((acc_rows, D), jnp.float32),
            pltpu.SemaphoreType.DMA,
            pltpu.SemaphoreType.DMA,
            pltpu.SemaphoreType.DMA,
            pltpu.SemaphoreType.DMA,
        ],
    )
    def ak(x_hbm, ss_hbm, ls_hbm, offs_hbm, z_hbm, agg_hbm,
           offs_v, srcb, ldstb, buf0, buf1, acc, sem0, sem1, semi, sema):
        c = lax.axis_index("c")
        t = lax.axis_index("s")
        iota = _IOTA()
        pltpu.sync_copy(offs_hbm, offs_v)
        bufs = (buf0, buf1)
        sems = (sem0, sem1)
        rpt = _BK // _NS

        def _fix_window(i, lo, hi, row_base):
            p0 = (row_base + i) * 128
            for sub in range(8):
                pos = p0 + sub * _L + iota
                valid = (pos >= lo) & (pos < hi)
                sv = srcb[i, pl.ds(sub * _L, _L)]
                lv = ldstb[i, pl.ds(sub * _L, _L)]
                srcb[i, pl.ds(sub * _L, _L)] = jnp.where(valid, sv, sub * _L + iota)
                ldstb[i, pl.ds(sub * _L, _L)] = jnp.where(valid, lv, _BK)

        @pl.loop(0, _NB // 2)
        def _bucket(bi):
            b = bi * 2 + c

            @pl.when(t < _NS - 1)
            def _z0():
                pltpu.sync_copy(z_hbm.at[pl.ds(0, rpt)],
                                acc.at[pl.ds(t * rpt, rpt)])

            @pl.when(t == _NS - 1)
            def _z1():
                pltpu.sync_copy(z_hbm, acc.at[pl.ds((_NS - 1) * rpt, zrows)])

            plsc.subcore_barrier()

            lo = offs_v[pl.ds(b, _L)][0]
            hi = offs_v[pl.ds(b + 1, _L)][0]
            r0 = lax.shift_left(lax.shift_right_logical(lo, 10), 3)
            row_end = lax.shift_right_logical(hi + 127, 7)

            def _cond(row_base):
                return row_base < row_end

            def _body(row_base):
                row_base = pl.multiple_of(row_base, 8)
                i0 = pltpu.async_copy(ss_hbm.at[pl.ds(row_base, 8)], srcb, semi)
                i1 = pltpu.async_copy(ls_hbm.at[pl.ds(row_base, 8)], ldstb, semi)
                i0.wait()
                i1.wait()
                _fix_window(0, lo, hi, row_base)
                g = pltpu.async_copy(x_hbm.at[srcb.at[0]], bufs[0], sems[0])
                descs = [g]
                for i in range(8):
                    if i < 7:
                        _fix_window(i + 1, lo, hi, row_base)
                        descs.append(pltpu.async_copy(
                            x_hbm.at[srcb.at[i + 1]],
                            bufs[(i + 1) % 2], sems[(i + 1) % 2]))
                    descs[i].wait()
                    pltpu.async_copy(bufs[i % 2], acc.at[ldstb.at[i]],
                                     sema, add=True).wait()
                return row_base + _NS * 8

            lax.while_loop(_cond, _body, r0 + t * 8)
            plsc.subcore_barrier()
            pltpu.sync_copy(acc.at[pl.ds(t * rpt, rpt)],
                            agg_hbm.at[pl.ds(b * _BK + t * rpt, rpt)])
            plsc.subcore_barrier()

    return ak(x, ss2d, ls2d, offs, zeros)


def _lg_scatter_add(x, prep):
    ss2d, ls2d, offs = prep
    return _lg_aggregate(x, ss2d, ls2d, offs)


def _lg_prepare(src, dst):
    counts = _lg_histogram(dst)
    ss, ls, offs = _lg_permute(src, dst, counts)
    return ss.reshape(-1, 128), ls.reshape(-1, 128), offs


def _gin_mlp_body(x_ref, agg_ref, w1_ref, bb_ref, s_ref, w2_ref, b2_ref, o_ref):
    h = x_ref[...] + agg_ref[0]
    for p in range(1, agg_ref.shape[0]):
        h = h + agg_ref[p]
    t = jnp.dot(h, w1_ref[...], preferred_element_type=jnp.float32)
    t = jax.nn.relu(t * s_ref[...] + bb_ref[...])
    t = jnp.dot(t, w2_ref[...], preferred_element_type=jnp.float32)
    o_ref[...] = jax.nn.relu(t + b2_ref[...])


def _gin_mlp(x, agg, p, block_rows=1024):
    r = x.shape[0]
    nparts = agg.shape[0]
    s = (p["g"] * _BN_INV).reshape(1, D)
    bb = (p["b1"] * p["g"] * _BN_INV + p["be"]).reshape(1, D)
    b2 = p["b2"].reshape(1, D)
    grid = (pl.cdiv(r, block_rows),)
    return pl.pallas_call(
        _gin_mlp_body,
        grid=grid,
        in_specs=[
            pl.BlockSpec((block_rows, D), lambda i: (i, 0)),
            pl.BlockSpec((nparts, block_rows, D), lambda i: (0, i, 0)),
            pl.BlockSpec((D, D), lambda i: (0, 0)),
            pl.BlockSpec((1, D), lambda i: (0, 0)),
            pl.BlockSpec((1, D), lambda i: (0, 0)),
            pl.BlockSpec((D, D), lambda i: (0, 0)),
            pl.BlockSpec((1, D), lambda i: (0, 0)),
        ],
        out_specs=pl.BlockSpec((block_rows, D), lambda i: (i, 0)),
        out_shape=jax.ShapeDtypeStruct((r, D), jnp.float32),
    )(x, agg, p["w1"], bb, s, p["w2"], b2)


def _pair_gather(x0, ep_t):
    e = ep_t.shape[1]
    ep_flat = ep_t.reshape(2 * e)
    rows_per = (2 * e) // _NW
    kwin = 80
    nwin = rows_per // kwin
    mesh = plsc.VectorSubcoreMesh(core_axis_name="c", subcore_axis_name="s")

    @functools.partial(
        pl.kernel,
        out_type=jax.ShapeDtypeStruct((2, e, D), jnp.float32),
        mesh=mesh,
        scratch_types=[
            pltpu.VMEM((rows_per,), jnp.int32),
            pltpu.VMEM((kwin, D), jnp.float32),
            pltpu.SemaphoreType.DMA,
        ],
    )
    def gk(x_hbm, ep_hbm, out_hbm, idx_v, buf, sem):
        w = lax.axis_index("c") * _NS + lax.axis_index("s")
        part = w // _NS
        rowbase = (w % _NS) * rows_per
        pltpu.sync_copy(ep_hbm.at[pl.ds(w * rows_per, rows_per)], idx_v)

        @pl.loop(0, nwin)
        def _win(j):
            idx = idx_v.at[pl.ds(j * kwin, kwin)]
            pltpu.async_copy(x_hbm.at[idx], buf, sem).wait()
            pltpu.sync_copy(buf, out_hbm.at[part, pl.ds(rowbase + j * kwin, kwin)])

    return gk(x0, ep_flat)


def _pair_mlp_body(g_ref, w1_ref, bb_ref, s_ref, w2_ref, b2_ref, o_ref):
    ta = jnp.dot(g_ref[0], w1_ref[...], preferred_element_type=jnp.float32)
    tb = jnp.dot(g_ref[1], w1_ref[...], preferred_element_type=jnp.float32)
    ta = jax.nn.relu(ta * s_ref[...] + bb_ref[...])
    tb = jax.nn.relu(tb * s_ref[...] + bb_ref[...])
    t = jnp.dot(ta + tb, w2_ref[...], preferred_element_type=jnp.float32)
    o_ref[...] = t + b2_ref[...]


def _pair_mlp(g, p, block_rows=1024):
    r = g.shape[1]
    s = (p["g"] * _BN_INV).reshape(1, D)
    bb = (p["b1"] * p["g"] * _BN_INV + p["be"]).reshape(1, D)
    b2 = (2.0 * p["b2"]).reshape(1, D)
    grid = (pl.cdiv(r, block_rows),)
    return pl.pallas_call(
        _pair_mlp_body,
        grid=grid,
        in_specs=[
            pl.BlockSpec((2, block_rows, D), lambda i: (0, i, 0)),
            pl.BlockSpec((D, D), lambda i: (0, 0)),
            pl.BlockSpec((1, D), lambda i: (0, 0)),
            pl.BlockSpec((1, D), lambda i: (0, 0)),
            pl.BlockSpec((D, D), lambda i: (0, 0)),
            pl.BlockSpec((1, D), lambda i: (0, 0)),
        ],
        out_specs=pl.BlockSpec((block_rows, D), lambda i: (i, 0)),
        out_shape=jax.ShapeDtypeStruct((r, D), jnp.float32),
    )(g, p["w1"], bb, s, p["w2"], b2)


def _linear_body(x_ref, w_ref, b_ref, o_ref):
    t = jnp.dot(x_ref[...], w_ref[...], preferred_element_type=jnp.float32)
    o_ref[...] = t + b_ref[...]


def _linear(x, w, b, block_rows=1024):
    r = x.shape[0]
    grid = (pl.cdiv(r, block_rows),)
    return pl.pallas_call(
        _linear_body,
        grid=grid,
        in_specs=[
            pl.BlockSpec((block_rows, D), lambda i: (i, 0)),
            pl.BlockSpec((D, D), lambda i: (0, 0)),
            pl.BlockSpec((1, D), lambda i: (0, 0)),
        ],
        out_specs=pl.BlockSpec((block_rows, D), lambda i: (i, 0)),
        out_shape=jax.ShapeDtypeStruct((r, D), jnp.float32),
    )(x, w, b.reshape(1, D))


def kernel(x_original, edge_index, edge_index_original, edge_pairs, params):
    ei2 = edge_index.astype(jnp.int32)
    eio = edge_index_original.astype(jnp.int32)
    ep = edge_pairs.astype(jnp.int32)

    x0 = x_original
    for p in params["init"]:
        agg = _scatter_add_small(x0, eio[0], eio[1], x0.shape[0])
        x0 = _gin_mlp(x0, agg, p)

    g = _pair_gather(x0, ep.T)
    x = _pair_mlp(g, params["mlp"])

    prep = _lg_prepare(ei2[0], ei2[1])
    for p in params["gin"]:
        agg = _lg_scatter_add(x, prep)
        x = _gin_mlp(x, agg[None], p)

    return _linear(x, params["wo"], params["bo"])
